# baseline (speedup 1.0000x reference)
import jax
import jax.numpy as jnp
from jax import lax
from jax.experimental import pallas as pl
from jax.experimental.pallas import tpu as pltpu
from jax.experimental.pallas import tpu_sc as plsc

MAX_REL = 128
NUM_HEADS = 16
SEQ_LEN = 2048
NPHASE = 16
NTILE = SEQ_LEN // 128
BAND_W0 = 14
BAND_TILES = 5
CONST_TILES = 13

NUM_CORES = 2
NUM_SUBCORES = 16
PHASES_PER_WORKER = 8
INFLIGHT_SLABS = 6


def _bias_body(band_hbm, const_hbm, out_hbm, band_v, const_v, sem):
    cid = lax.axis_index("c")
    sid = lax.axis_index("s")
    wid = sid * NUM_CORES + cid
    head = wid // 2
    pbase = (wid % 2) * PHASES_PER_WORKER

    for e in range(PHASES_PER_WORKER):
        pltpu.sync_copy(band_hbm.at[head, pbase + e], band_v.at[e])
    pltpu.sync_copy(const_hbm.at[head], const_v)

    def slab_dmas(q, e):
        a = 16 * q + pbase + e
        row = pl.ds(pl.multiple_of(8 * a, 8), 8)
        tb = min(max(q - 1, 0), CONST_TILES)
        widx = tb + NTILE - q - BAND_W0
        dmas = []
        if tb > 0:
            dmas.append(pltpu.make_async_copy(
                const_v.at[0, :, pl.ds(0, 128 * tb)],
                out_hbm.at[head, row, pl.ds(0, 128 * tb)],
                sem,
            ))
        dmas.append(pltpu.make_async_copy(
            band_v.at[e, :, pl.ds(128 * widx, 384)],
            out_hbm.at[head, row, pl.ds(128 * tb, 384)],
            sem,
        ))
        if tb < CONST_TILES:
            n = CONST_TILES - tb
            dmas.append(pltpu.make_async_copy(
                const_v.at[1, :, pl.ds(0, 128 * n)],
                out_hbm.at[head, row, pl.ds(128 * (tb + 3), 128 * n)],
                sem,
            ))
        return dmas

    K = INFLIGHT_SLABS

    def drain(q, e):
        for d in slab_dmas(q, e):
            d.wait()

    for q in range(NTILE):
        def body(e, carry, q=q):
            for d in slab_dmas(q, e):
                d.start()

            @pl.when(e >= K)
            def _():
                drain(q, e - K)

            if q > 0:

                @pl.when(e < K)
                def _():
                    drain(q - 1, e + PHASES_PER_WORKER - K)

            return carry

        lax.fori_loop(0, PHASES_PER_WORKER, body, 0, unroll=2)
    for e in range(PHASES_PER_WORKER - K, PHASES_PER_WORKER):
        drain(NTILE - 1, e)


@jax.jit
def _bias_sc(band, const):
    mesh = plsc.VectorSubcoreMesh(core_axis_name="c", subcore_axis_name="s")
    return pl.kernel(
        _bias_body,
        out_type=jax.ShapeDtypeStruct((NUM_HEADS, SEQ_LEN, SEQ_LEN), jnp.float32),
        mesh=mesh,
        scratch_types=[
            pltpu.VMEM((PHASES_PER_WORKER, 8, 128 * BAND_TILES), jnp.float32),
            pltpu.VMEM((2, 8, 128 * CONST_TILES), jnp.float32),
            pltpu.SemaphoreType.DMA,
        ],
    )(band, const)


def kernel(seq_len, W):
    del seq_len
    pad = 136
    lo = pad + SEQ_LEN - 1 - MAX_REL
    wt = W.T
    fpt = jnp.concatenate(
        [
            jnp.broadcast_to(wt[:, :1], (NUM_HEADS, lo)),
            wt,
            jnp.broadcast_to(wt[:, -1:], (NUM_HEADS, 264)),
        ],
        axis=1,
    )
    s8 = jnp.stack(
        [fpt[:, pad + 1664 - r : pad + 2432 - r] for r in range(8)], axis=1
    )
    band = jnp.stack(
        [s8[:, :, 127 - 8 * p : 767 - 8 * p] for p in range(NPHASE)], axis=1
    )
    const = jnp.broadcast_to(
        jnp.stack([W[0], W[-1]], 0).T[:, :, None, None],
        (NUM_HEADS, 2, 8, 128 * CONST_TILES),
    )
    return _bias_sc(band, const)


# speedup vs baseline: 1.0436x; 1.0436x over previous
import jax
import jax.numpy as jnp
from jax import lax
from jax.experimental import pallas as pl
from jax.experimental.pallas import tpu as pltpu
from jax.experimental.pallas import tpu_sc as plsc

MAX_REL = 128
NUM_HEADS = 16
SEQ_LEN = 2048
NPHASE = 16
NTILE = SEQ_LEN // 128
BAND_W0 = 14
BAND_TILES = 5
CONST_TILES = 13

NUM_CORES = 2
NUM_SUBCORES = 16
PHASES_PER_WORKER = 8
INFLIGHT_SLABS = 6


def _bias_body(band_hbm, const_hbm, out_hbm, band_v, const_v, sem):
    cid = lax.axis_index("c")
    sid = lax.axis_index("s")
    wid = sid * NUM_CORES + cid
    head = wid // 2
    pbase = (wid % 2) * PHASES_PER_WORKER

    stage_band = pltpu.make_async_copy(
        band_hbm.at[head, pl.ds(pbase, PHASES_PER_WORKER)], band_v, sem
    )
    stage_const = pltpu.make_async_copy(const_hbm.at[head], const_v, sem)
    stage_band.start()
    stage_const.start()
    stage_band.wait()
    stage_const.wait()

    def slab_dmas(q, e):
        a = 16 * q + pbase + e
        row = pl.ds(pl.multiple_of(8 * a, 8), 8)
        tb = min(max(q - 1, 0), CONST_TILES)
        widx = tb + NTILE - q - BAND_W0
        dmas = []
        if tb > 0:
            dmas.append(pltpu.make_async_copy(
                const_v.at[0, :, pl.ds(0, 128 * tb)],
                out_hbm.at[head, row, pl.ds(0, 128 * tb)],
                sem,
            ))
        dmas.append(pltpu.make_async_copy(
            band_v.at[e, :, pl.ds(128 * widx, 384)],
            out_hbm.at[head, row, pl.ds(128 * tb, 384)],
            sem,
        ))
        if tb < CONST_TILES:
            n = CONST_TILES - tb
            dmas.append(pltpu.make_async_copy(
                const_v.at[1, :, pl.ds(0, 128 * n)],
                out_hbm.at[head, row, pl.ds(128 * (tb + 3), 128 * n)],
                sem,
            ))
        return dmas

    K = INFLIGHT_SLABS

    def drain(q, e):
        for d in slab_dmas(q, e):
            d.wait()

    for q in range(NTILE):
        def body(e, carry, q=q):
            for d in slab_dmas(q, e):
                d.start()

            @pl.when(e >= K)
            def _():
                drain(q, e - K)

            if q > 0:

                @pl.when(e < K)
                def _():
                    drain(q - 1, e + PHASES_PER_WORKER - K)

            return carry

        lax.fori_loop(0, PHASES_PER_WORKER, body, 0, unroll=2)
    for e in range(PHASES_PER_WORKER - K, PHASES_PER_WORKER):
        drain(NTILE - 1, e)


@jax.jit
def _bias_sc(band, const):
    mesh = plsc.VectorSubcoreMesh(core_axis_name="c", subcore_axis_name="s")
    return pl.kernel(
        _bias_body,
        out_type=jax.ShapeDtypeStruct((NUM_HEADS, SEQ_LEN, SEQ_LEN), jnp.float32),
        mesh=mesh,
        scratch_types=[
            pltpu.VMEM((PHASES_PER_WORKER, 8, 128 * BAND_TILES), jnp.float32),
            pltpu.VMEM((2, 8, 128 * CONST_TILES), jnp.float32),
            pltpu.SemaphoreType.DMA,
        ],
    )(band, const)


def kernel(seq_len, W):
    del seq_len
    pad = 136
    lo = pad + SEQ_LEN - 1 - MAX_REL
    wt = W.T
    fpt = jnp.concatenate(
        [
            jnp.broadcast_to(wt[:, :1], (NUM_HEADS, lo)),
            wt,
            jnp.broadcast_to(wt[:, -1:], (NUM_HEADS, 264)),
        ],
        axis=1,
    )
    s8 = jnp.stack(
        [fpt[:, pad + 1664 - r : pad + 2432 - r] for r in range(8)], axis=1
    )
    band = jnp.stack(
        [s8[:, :, 127 - 8 * p : 767 - 8 * p] for p in range(NPHASE)], axis=1
    )
    const = jnp.broadcast_to(
        jnp.stack([W[0], W[-1]], 0).T[:, :, None, None],
        (NUM_HEADS, 2, 8, 128 * CONST_TILES),
    )
    return _bias_sc(band, const)
